# baseline (speedup 1.0000x reference)
import functools

import jax
import jax.numpy as jnp
from jax.experimental import pallas as pl
from jax.experimental.pallas import tpu as pltpu

H = 64
W = 64
G = 5000
FRONT_K = 8
ALPHA_THRESHOLD = 1.0 / 255.0

CHUNK = 256
PIXB = 512
G_PAD = ((G + CHUNK - 1) // CHUNK) * CHUNK
NCHUNK = G_PAD // CHUNK
NPIX = H * W
NBLK = NPIX // PIXB


def _raster_kernel(packed_ref, out_ref):
    lane = jax.lax.broadcasted_iota(jnp.int32, (1, PIXB), 1)

    rowi = jax.lax.broadcasted_iota(jnp.int32, (CHUNK, CHUNK), 0)
    coli = jax.lax.broadcasted_iota(jnp.int32, (CHUNK, CHUNK), 1)
    tri = (rowi >= coli).astype(jnp.float32)

    def do_block(blk, _):
        p = blk * PIXB + lane
        px = (p % W).astype(jnp.float32) + 0.5
        py = (p // W).astype(jnp.float32) + 0.5

        def body(carry):
            i, count, T, rgb = carry
            chunk = packed_ref[pl.ds(i * CHUNK, CHUNK), :]
            mx = chunk[:, 0:1]
            my = chunk[:, 1:2]
            ca = chunk[:, 2:3]
            cb = chunk[:, 3:4]
            cc = chunk[:, 4:5]
            op = chunk[:, 5:6]
            cols = chunk[:, 6:14]

            dx = px - mx
            dy = py - my
            power = (ca * dx + cb * dy) * dx + (cc * dy) * dy
            alpha = jnp.minimum(0.999, op * jnp.exp(-power))
            visible = (alpha > ALPHA_THRESHOLD) & (power >= 0.0)
            vis_f = visible.astype(jnp.float32)
            rank_in = jax.lax.dot_general(
                tri, vis_f, (((1,), (0,)), ((), ())),
                preferred_element_type=jnp.float32)
            keep = visible & ((count + rank_in) <= float(FRONT_K))
            eff = jnp.where(keep, alpha, 0.0)
            log_t = jnp.log(1.0 - eff)
            cum = jax.lax.dot_general(
                tri, log_t, (((1,), (0,)), ((), ())),
                preferred_element_type=jnp.float32)
            w = eff * T * jnp.exp(cum - log_t)
            rgb = rgb + jax.lax.dot_general(
                w, cols, (((0,), (0,)), ((), ())),
                preferred_element_type=jnp.float32)
            T = T * jnp.exp(cum[CHUNK - 1:CHUNK, :])
            count = count + rank_in[CHUNK - 1:CHUNK, :]
            return i + 1, count, T, rgb

        def cond(carry):
            i, count, _, _ = carry
            return (i < NCHUNK) & (jnp.min(count) < float(FRONT_K))

        init = (jnp.int32(0),
                jnp.zeros((1, PIXB), jnp.float32),
                jnp.ones((1, PIXB), jnp.float32),
                jnp.zeros((PIXB, 8), jnp.float32))
        _, _, _, rgb = jax.lax.while_loop(cond, body, init)
        out_ref[pl.ds(blk * PIXB, PIXB), :] = rgb[:, :3]
        return 0

    jax.lax.fori_loop(0, NBLK, do_block, 0)


@functools.partial(jax.jit, static_argnames=())
def _run(means2d, conics, colors, opacities, depths):
    ops = jax.lax.sort(
        (depths[0], means2d[0, :, 0], means2d[0, :, 1],
         conics[0, :, 0] * 0.5, conics[0, :, 1], conics[0, :, 2] * 0.5,
         opacities[0], colors[0, :, 0], colors[0, :, 1], colors[0, :, 2]),
        num_keys=1, is_stable=True)
    packed_s = jnp.stack(ops[1:], axis=1)
    packed_s = jnp.pad(packed_s, ((0, G_PAD - G), (0, 7)))

    out = pl.pallas_call(
        _raster_kernel,
        out_shape=jax.ShapeDtypeStruct((NPIX, 3), jnp.float32),
    )(packed_s)
    return out.reshape(1, H, W, 3)


def kernel(means2d, conics, colors, opacities, depths):
    return _run(means2d, conics, colors, opacities, depths)


# speedup vs baseline: 1.0611x; 1.0611x over previous
import functools

import jax
import jax.numpy as jnp
from jax.experimental import pallas as pl
from jax.experimental.pallas import tpu as pltpu

H = 64
W = 64
G = 5000
FRONT_K = 8
ALPHA_THRESHOLD = 1.0 / 255.0

CHUNK = 256
PIXB = 512
G_PAD = ((G + CHUNK - 1) // CHUNK) * CHUNK
NCHUNK = G_PAD // CHUNK
NPIX = H * W
NBLK = NPIX // PIXB


def _raster_kernel(packed_ref, out_ref):
    lane = jax.lax.broadcasted_iota(jnp.int32, (1, PIXB), 1)

    rowi = jax.lax.broadcasted_iota(jnp.int32, (CHUNK, CHUNK), 0)
    coli = jax.lax.broadcasted_iota(jnp.int32, (CHUNK, CHUNK), 1)
    tri = (rowi >= coli).astype(jnp.float32)

    def do_block(blk, _):
        p = blk * PIXB + lane
        px = (p % W).astype(jnp.float32) + 0.5
        py = (p // W).astype(jnp.float32) + 0.5

        def one_chunk(i, count, T, rgb):
            chunk = packed_ref[pl.ds(i * CHUNK, CHUNK), :]
            mx = chunk[:, 0:1]
            my = chunk[:, 1:2]
            ca = chunk[:, 2:3]
            cb = chunk[:, 3:4]
            cc = chunk[:, 4:5]
            op = chunk[:, 5:6]
            cols = chunk[:, 6:14]

            dx = px - mx
            dy = py - my
            power = (ca * dx + cb * dy) * dx + (cc * dy) * dy
            alpha = jnp.minimum(0.999, op * jnp.exp(-power))
            visible = (alpha > ALPHA_THRESHOLD) & (power >= 0.0)
            vis_f = visible.astype(jnp.float32)
            rank_in = jax.lax.dot_general(
                tri, vis_f, (((1,), (0,)), ((), ())),
                preferred_element_type=jnp.float32)
            keep = visible & ((count + rank_in) <= float(FRONT_K))
            eff = jnp.where(keep, alpha, 0.0)
            log_t = jnp.log(1.0 - eff)
            cum = jax.lax.dot_general(
                tri, log_t, (((1,), (0,)), ((), ())),
                preferred_element_type=jnp.float32)
            w = eff * T * jnp.exp(cum - log_t)
            rgb = rgb + jax.lax.dot_general(
                w, cols, (((0,), (0,)), ((), ())),
                preferred_element_type=jnp.float32)
            T = T * jnp.exp(cum[CHUNK - 1:CHUNK, :])
            count = count + rank_in[CHUNK - 1:CHUNK, :]
            return count, T, rgb

        def body(carry):
            i, count, T, rgb = carry
            count, T, rgb = one_chunk(i, count, T, rgb)
            count, T, rgb = one_chunk(i + 1, count, T, rgb)
            return i + 2, count, T, rgb

        def cond(carry):
            i, count, _, _ = carry
            return (i < NCHUNK) & (jnp.min(count) < float(FRONT_K))

        init = (jnp.int32(0),
                jnp.zeros((1, PIXB), jnp.float32),
                jnp.ones((1, PIXB), jnp.float32),
                jnp.zeros((PIXB, 8), jnp.float32))
        _, _, _, rgb = jax.lax.while_loop(cond, body, init)
        out_ref[pl.ds(blk * PIXB, PIXB), :] = rgb[:, :3]
        return 0

    jax.lax.fori_loop(0, NBLK, do_block, 0)


@functools.partial(jax.jit, static_argnames=())
def _run(means2d, conics, colors, opacities, depths):
    ops = jax.lax.sort(
        (depths[0], means2d[0, :, 0], means2d[0, :, 1],
         conics[0, :, 0] * 0.5, conics[0, :, 1], conics[0, :, 2] * 0.5,
         opacities[0], colors[0, :, 0], colors[0, :, 1], colors[0, :, 2]),
        num_keys=1, is_stable=True)
    packed_s = jnp.stack(ops[1:], axis=1)
    packed_s = jnp.pad(packed_s, ((0, G_PAD - G), (0, 7)))

    out = pl.pallas_call(
        _raster_kernel,
        out_shape=jax.ShapeDtypeStruct((NPIX, 3), jnp.float32),
    )(packed_s)
    return out.reshape(1, H, W, 3)


def kernel(means2d, conics, colors, opacities, depths):
    return _run(means2d, conics, colors, opacities, depths)
